# Initial kernel scaffold; baseline (speedup 1.0000x reference)
import jax
import jax.numpy as jnp
from jax.experimental import pallas as pl


def kernel(token_ids, emb_table, W, b):
    raise NotImplementedError("write your pallas kernel here")


# speedup vs baseline: 3.5386x; 3.5386x over previous
import functools

import jax
import jax.numpy as jnp
from jax import lax
from jax.experimental import pallas as pl
from jax.experimental.pallas import tpu as pltpu
from jax.experimental.pallas import tpu_sc as plsc

VOCAB = 100000
EMB_DIM = 128
NUM_CLASSES = 64


_ROWS_PER_BLOCK = 800


def _project_body(emb_ref, w_ref, b_ref, out_ref):
    out_ref[...] = (
        jnp.dot(emb_ref[...], w_ref[...], preferred_element_type=jnp.float32)
        + b_ref[...]
    )


def _project(emb_table, W, b2d):
    n_blocks = VOCAB // _ROWS_PER_BLOCK
    return pl.pallas_call(
        _project_body,
        grid=(n_blocks,),
        in_specs=[
            pl.BlockSpec((_ROWS_PER_BLOCK, EMB_DIM), lambda i: (i, 0)),
            pl.BlockSpec((EMB_DIM, NUM_CLASSES), lambda i: (0, 0)),
            pl.BlockSpec((1, NUM_CLASSES), lambda i: (0, 0)),
        ],
        out_specs=pl.BlockSpec((_ROWS_PER_BLOCK, NUM_CLASSES), lambda i: (i, 0)),
        out_shape=jax.ShapeDtypeStruct((VOCAB, NUM_CLASSES), jnp.float32),
    )(emb_table, W, b2d)


_CHUNK = 128


def _make_gather(n_ids, nw):
    ids_per_w = n_ids // nw
    n_chunks = ids_per_w // _CHUNK
    assert n_chunks % 2 == 0
    mesh = plsc.VectorSubcoreMesh(core_axis_name="c", subcore_axis_name="s")
    nc = mesh.num_cores

    @functools.partial(
        pl.kernel,
        mesh=mesh,
        out_type=jax.ShapeDtypeStruct((n_ids, NUM_CLASSES), jnp.float32),
        scratch_types=[
            pltpu.VMEM((n_chunks, _CHUNK), jnp.int32),
            pltpu.VMEM((_CHUNK, NUM_CLASSES), jnp.float32),
            pltpu.VMEM((_CHUNK, NUM_CLASSES), jnp.float32),
            pltpu.SemaphoreType.DMA,
            pltpu.SemaphoreType.DMA,
        ],
        compiler_params=pltpu.CompilerParams(use_tc_tiling_on_sc=False),
    )
    def gather_k(ids_hbm, p_hbm, out_hbm, idx_v, buf0, buf1, sem0, sem1):
        wid = lax.axis_index("s") * nc + lax.axis_index("c")
        base = wid * ids_per_w
        pltpu.sync_copy(ids_hbm.at[wid], idx_v)
        pltpu.async_copy(p_hbm.at[idx_v.at[0]], buf0, sem0)

        def body(i, carry):
            j = 2 * i
            pltpu.async_copy(p_hbm.at[idx_v.at[j + 1]], buf1, sem1)
            pltpu.make_async_copy(p_hbm.at[idx_v.at[j]], buf0, sem0).wait()
            pltpu.sync_copy(buf0, out_hbm.at[pl.ds(base + j * _CHUNK, _CHUNK)])

            @pl.when(j + 2 < n_chunks)
            def _():
                pltpu.async_copy(p_hbm.at[idx_v.at[j + 2]], buf0, sem0)

            pltpu.make_async_copy(p_hbm.at[idx_v.at[j + 1]], buf1, sem1).wait()
            pltpu.sync_copy(
                buf1, out_hbm.at[pl.ds(base + (j + 1) * _CHUNK, _CHUNK)]
            )
            return carry

        lax.fori_loop(0, n_chunks // 2, body, 0)

    return gather_k


def kernel(token_ids, emb_table, W, b):
    B, L = token_ids.shape
    n_ids = B * L
    info = plsc.get_sparse_core_info()
    nw = info.num_cores * info.num_subcores

    proj = _project(emb_table, W, b.reshape(1, NUM_CLASSES))

    ids3d = token_ids.reshape(nw, (n_ids // nw) // _CHUNK, _CHUNK)
    flat = _make_gather(n_ids, nw)(ids3d.astype(jnp.int32), proj)
    return flat.reshape(B, L, NUM_CLASSES)


# speedup vs baseline: 3.6858x; 1.0416x over previous
import functools

import jax
import jax.numpy as jnp
from jax import lax
from jax.experimental import pallas as pl
from jax.experimental.pallas import tpu as pltpu
from jax.experimental.pallas import tpu_sc as plsc

VOCAB = 100000
EMB_DIM = 128
NUM_CLASSES = 64


_ROWS_PER_BLOCK = 800


def _project_body(emb_ref, w_ref, b_ref, out_ref):
    out_ref[...] = (
        jnp.dot(emb_ref[...], w_ref[...], preferred_element_type=jnp.float32)
        + b_ref[...]
    )


def _project(emb_table, W2, b2):
    n_blocks = VOCAB // _ROWS_PER_BLOCK
    return pl.pallas_call(
        _project_body,
        grid=(n_blocks,),
        in_specs=[
            pl.BlockSpec((_ROWS_PER_BLOCK, EMB_DIM), lambda i: (i, 0)),
            pl.BlockSpec((EMB_DIM, 2 * NUM_CLASSES), lambda i: (0, 0)),
            pl.BlockSpec((1, 2 * NUM_CLASSES), lambda i: (0, 0)),
        ],
        out_specs=pl.BlockSpec(
            (_ROWS_PER_BLOCK, 2 * NUM_CLASSES), lambda i: (i, 0)
        ),
        out_shape=jax.ShapeDtypeStruct((VOCAB, 2 * NUM_CLASSES), jnp.float32),
    )(emb_table, W2, b2)


_CHUNK = 128


def _make_gather(n_ids, nw):
    ids_per_w = n_ids // nw
    n_chunks = ids_per_w // _CHUNK
    assert n_chunks % 2 == 0
    mesh = plsc.VectorSubcoreMesh(core_axis_name="c", subcore_axis_name="s")
    nc = mesh.num_cores

    @functools.partial(
        pl.kernel,
        mesh=mesh,
        out_type=jax.ShapeDtypeStruct((n_ids, NUM_CLASSES), jnp.float32),
        scratch_types=[
            pltpu.VMEM((n_chunks, _CHUNK), jnp.int32),
            pltpu.VMEM((_CHUNK, 2 * NUM_CLASSES), jnp.float32),
            pltpu.VMEM((_CHUNK, 2 * NUM_CLASSES), jnp.float32),
            pltpu.VMEM((_CHUNK, NUM_CLASSES), jnp.float32),
            pltpu.VMEM((_CHUNK, NUM_CLASSES), jnp.float32),
            pltpu.SemaphoreType.DMA,
            pltpu.SemaphoreType.DMA,
        ],
    )
    def gather_k(
        ids_hbm, p_hbm, out_hbm, idx_v, gbuf0, gbuf1, obuf0, obuf1, sem0, sem1
    ):
        wid = lax.axis_index("s") * nc + lax.axis_index("c")
        base = wid * ids_per_w
        pltpu.sync_copy(ids_hbm.at[wid], idx_v)
        pltpu.async_copy(p_hbm.at[idx_v.at[0]], gbuf0, sem0)
        pltpu.async_copy(p_hbm.at[idx_v.at[1]], gbuf1, sem1)

        def compact(gbuf, obuf):
            def row(r, c):
                for k in range(NUM_CLASSES // 16):
                    obuf[r, pl.ds(16 * k, 16)] = gbuf[r, pl.ds(16 * k, 16)]
                return c

            lax.fori_loop(0, _CHUNK, row, 0, unroll=8)

        def body(i, carry):
            j = 2 * i
            pltpu.make_async_copy(p_hbm.at[idx_v.at[j]], gbuf0, sem0).wait()
            compact(gbuf0, obuf0)

            @pl.when(j + 2 < n_chunks)
            def _():
                pltpu.async_copy(p_hbm.at[idx_v.at[j + 2]], gbuf0, sem0)

            pltpu.sync_copy(
                obuf0, out_hbm.at[pl.ds(base + j * _CHUNK, _CHUNK)]
            )
            pltpu.make_async_copy(p_hbm.at[idx_v.at[j + 1]], gbuf1, sem1).wait()
            compact(gbuf1, obuf1)

            @pl.when(j + 3 < n_chunks)
            def _():
                pltpu.async_copy(p_hbm.at[idx_v.at[j + 3]], gbuf1, sem1)

            pltpu.sync_copy(
                obuf1, out_hbm.at[pl.ds(base + (j + 1) * _CHUNK, _CHUNK)]
            )
            return carry

        lax.fori_loop(0, n_chunks // 2, body, 0)

    return gather_k


def kernel(token_ids, emb_table, W, b):
    B, L = token_ids.shape
    n_ids = B * L
    info = plsc.get_sparse_core_info()
    nw = info.num_cores * info.num_subcores

    W2 = jnp.concatenate([W, W], axis=1)
    b2 = jnp.concatenate([b, b]).reshape(1, 2 * NUM_CLASSES)
    proj = _project(emb_table, W2, b2)

    ids3d = token_ids.reshape(nw, (n_ids // nw) // _CHUNK, _CHUNK)
    flat = _make_gather(n_ids, nw)(ids3d.astype(jnp.int32), proj)
    return flat.reshape(B, L, NUM_CLASSES)


# speedup vs baseline: 3.8327x; 1.0399x over previous
import functools

import jax
import jax.numpy as jnp
from jax import lax
from jax.experimental import pallas as pl
from jax.experimental.pallas import tpu as pltpu
from jax.experimental.pallas import tpu_sc as plsc

VOCAB = 100000
EMB_DIM = 128
NUM_CLASSES = 64


_ROWS_PER_BLOCK = 4000


def _project_body(emb_ref, w_ref, b_ref, out_ref):
    out_ref[...] = (
        jnp.dot(emb_ref[...], w_ref[...], preferred_element_type=jnp.float32)
        + b_ref[...]
    )


def _project(emb_table, W2, b2):
    n_blocks = VOCAB // _ROWS_PER_BLOCK
    return pl.pallas_call(
        _project_body,
        grid=(n_blocks,),
        in_specs=[
            pl.BlockSpec((_ROWS_PER_BLOCK, EMB_DIM), lambda i: (i, 0)),
            pl.BlockSpec((EMB_DIM, 2 * NUM_CLASSES), lambda i: (0, 0)),
            pl.BlockSpec((1, 2 * NUM_CLASSES), lambda i: (0, 0)),
        ],
        out_specs=pl.BlockSpec(
            (_ROWS_PER_BLOCK, 2 * NUM_CLASSES), lambda i: (i, 0)
        ),
        out_shape=jax.ShapeDtypeStruct((VOCAB, 2 * NUM_CLASSES), jnp.float32),
    )(emb_table, W2, b2)


_CHUNK = 40


def _make_gather(B, L, nw):
    n_ids = B * L
    ids_per_w = n_ids // nw
    n_chunks = ids_per_w // _CHUNK
    assert n_chunks % 2 == 0
    mesh = plsc.VectorSubcoreMesh(core_axis_name="c", subcore_axis_name="s")
    nc = mesh.num_cores

    @functools.partial(
        pl.kernel,
        mesh=mesh,
        out_type=jax.ShapeDtypeStruct((B, L, NUM_CLASSES), jnp.float32),
        scratch_types=[
            pltpu.VMEM((n_chunks, _CHUNK), jnp.int32),
            pltpu.VMEM((_CHUNK, 2 * NUM_CLASSES), jnp.float32),
            pltpu.VMEM((_CHUNK, 2 * NUM_CLASSES), jnp.float32),
            pltpu.VMEM((_CHUNK, NUM_CLASSES), jnp.float32),
            pltpu.VMEM((_CHUNK, NUM_CLASSES), jnp.float32),
            pltpu.SemaphoreType.DMA,
            pltpu.SemaphoreType.DMA,
            pltpu.SemaphoreType.DMA,
            pltpu.SemaphoreType.DMA,
        ],
    )
    def gather_k(
        ids_hbm, p_hbm, out_hbm,
        idx_v, gbuf0, gbuf1, obuf0, obuf1, gsem0, gsem1, ssem0, ssem1,
    ):
        wid = lax.axis_index("s") * nc + lax.axis_index("c")
        base = wid * ids_per_w
        pltpu.sync_copy(ids_hbm.at[wid], idx_v)
        pltpu.async_copy(p_hbm.at[idx_v.at[0]], gbuf0, gsem0)
        pltpu.async_copy(p_hbm.at[idx_v.at[1]], gbuf1, gsem1)

        def compact(gbuf, obuf):
            def row(r, c):
                for k in range(NUM_CLASSES // 16):
                    obuf[r, pl.ds(16 * k, 16)] = gbuf[r, pl.ds(16 * k, 16)]
                return c

            lax.fori_loop(0, _CHUNK, row, 0, unroll=8)

        def out_slice(j):
            flat = base + j * _CHUNK
            return out_hbm.at[flat // L, pl.ds(flat % L, _CHUNK)]

        def half_step(i, j, gbuf, obuf, gsem, ssem):
            pltpu.make_async_copy(p_hbm.at[idx_v.at[j]], gbuf, gsem).wait()

            @pl.when(i > 0)
            def _():
                pltpu.make_async_copy(obuf, out_slice(j - 2), ssem).wait()

            compact(gbuf, obuf)

            @pl.when(j + 2 < n_chunks)
            def _():
                pltpu.async_copy(p_hbm.at[idx_v.at[j + 2]], gbuf, gsem)

            pltpu.async_copy(obuf, out_slice(j), ssem)

        def body(i, carry):
            j = 2 * i
            half_step(i, j, gbuf0, obuf0, gsem0, ssem0)
            half_step(i, j + 1, gbuf1, obuf1, gsem1, ssem1)
            return carry

        lax.fori_loop(0, n_chunks // 2, body, 0)
        pltpu.make_async_copy(obuf0, out_slice(n_chunks - 2), ssem0).wait()
        pltpu.make_async_copy(obuf1, out_slice(n_chunks - 1), ssem1).wait()

    return gather_k


def kernel(token_ids, emb_table, W, b):
    B, L = token_ids.shape
    n_ids = B * L
    info = plsc.get_sparse_core_info()
    nw = info.num_cores * info.num_subcores

    W2 = jnp.concatenate([W, W], axis=1)
    b2 = jnp.concatenate([b, b]).reshape(1, 2 * NUM_CLASSES)
    proj = _project(emb_table, W2, b2)

    ids3d = token_ids.reshape(nw, (n_ids // nw) // _CHUNK, _CHUNK)
    return _make_gather(B, L, nw)(ids3d.astype(jnp.int32), proj)
